# Initial kernel scaffold; baseline (speedup 1.0000x reference)
import jax
import jax.numpy as jnp
from jax.experimental import pallas as pl


def kernel(in_feat, edge_index, W_self1, W_neigh1, b1, W_self2, W_neigh2, b2):
    raise NotImplementedError("write your pallas kernel here")


# speedup vs baseline: 6.6784x; 6.6784x over previous
import functools

import jax
import jax.numpy as jnp
from jax import lax
from jax.experimental import pallas as pl
from jax.experimental.pallas import tpu as pltpu, tpu_sc as plsc

N = 10000
E = 320000
D = 128
H = 128

NC = 2
NS = 16
NW = NC * NS
EPW = E // NW
CH = 128
K = -(-EPW // CH)
EPW_PAD = K * CH
NPAD = 10240
RPC = NPAD // NS

_mesh = plsc.VectorSubcoreMesh(
    core_axis_name="c", subcore_axis_name="s", num_cores=NC, num_subcores=NS)


def _sc_agg_body(h_hbm, src_hbm, dst_hbm, zf_hbm, agg_out,
                 src_v, dst_v, rows_v, agg_sh, sem):
    c = lax.axis_index("c")
    s = lax.axis_index("s")

    pltpu.sync_copy(zf_hbm.at[pl.ds(s * RPC, RPC)], agg_sh.at[pl.ds(s * RPC, RPC)])
    pltpu.sync_copy(src_hbm.at[c].at[s], src_v)
    pltpu.sync_copy(dst_hbm.at[c].at[s], dst_v)
    plsc.subcore_barrier()

    def chunk(j, carry):
        pltpu.async_copy(h_hbm.at[src_v.at[j]], rows_v, sem).wait()
        pltpu.sync_copy(rows_v, agg_sh.at[dst_v.at[j]], add=True)
        return carry

    lax.fori_loop(0, K, chunk, 0)
    plsc.subcore_barrier()

    pltpu.sync_copy(agg_sh.at[pl.ds(s * RPC, RPC)],
                    agg_out.at[c].at[pl.ds(s * RPC, RPC)])


_sc_agg = pl.kernel(
    _sc_agg_body,
    out_type=jax.ShapeDtypeStruct((NC, NPAD, D), jnp.float32),
    mesh=_mesh,
    scratch_types=[
        pltpu.VMEM((K, CH), jnp.int32),
        pltpu.VMEM((K, CH), jnp.int32),
        pltpu.VMEM((CH, D), jnp.float32),
        pltpu.VMEM_SHARED((NPAD, D), jnp.float32),
        pltpu.SemaphoreType.DMA,
    ],
)


def _sc_cnt_body(dst_hbm, zf_hbm, ones_hbm, cnt_out,
                 dst_v, ones_v, cnt_sh):
    c = lax.axis_index("c")
    s = lax.axis_index("s")

    pltpu.sync_copy(zf_hbm.at[pl.ds(s * RPC, RPC)], cnt_sh.at[pl.ds(s * RPC, RPC)])
    pltpu.sync_copy(ones_hbm, ones_v)
    pltpu.sync_copy(dst_hbm.at[c].at[s], dst_v)
    plsc.subcore_barrier()

    def chunk(j, carry):
        pltpu.sync_copy(ones_v, cnt_sh.at[dst_v.at[j]], add=True)
        return carry

    lax.fori_loop(0, K, chunk, 0)
    plsc.subcore_barrier()

    pltpu.sync_copy(cnt_sh.at[pl.ds(s * RPC, RPC)],
                    cnt_out.at[c].at[pl.ds(s * RPC, RPC)])


_sc_cnt = pl.kernel(
    _sc_cnt_body,
    out_type=jax.ShapeDtypeStruct((NC, NPAD, D), jnp.float32),
    mesh=_mesh,
    scratch_types=[
        pltpu.VMEM((K, CH), jnp.int32),
        pltpu.VMEM((CH, D), jnp.float32),
        pltpu.VMEM_SHARED((NPAD, D), jnp.float32),
    ],
)


def _tc_layer_body(relu, h, p0, p1, c0, c1, ws, wn, b, out):
    deg = c0[...] + c1[...] + 1.0
    agg = (p0[...] + p1[...] + h[...]) / deg
    acc = (jnp.dot(h[...], ws[...], preferred_element_type=jnp.float32)
           + jnp.dot(agg, wn[...], preferred_element_type=jnp.float32)
           + b[...])
    out[...] = jnp.maximum(acc, 0.0) if relu else acc


def _tc_layer(h, p0, p1, c0, c1, W_self, W_neigh, b, relu):
    R = 1000
    grid = N // R
    row = lambda i: (i, 0)
    full = lambda i: (0, 0)
    return pl.pallas_call(
        functools.partial(_tc_layer_body, relu),
        grid=(grid,),
        in_specs=[
            pl.BlockSpec((R, D), row),
            pl.BlockSpec((R, D), row),
            pl.BlockSpec((R, D), row),
            pl.BlockSpec((R, 1), row),
            pl.BlockSpec((R, 1), row),
            pl.BlockSpec((D, H), full),
            pl.BlockSpec((D, H), full),
            pl.BlockSpec((1, H), full),
        ],
        out_specs=pl.BlockSpec((R, H), row),
        out_shape=jax.ShapeDtypeStruct((N, H), jnp.float32),
    )(h, p0, p1, c0, c1, W_self, W_neigh, b)


def kernel(in_feat, edge_index, W_self1, W_neigh1, b1, W_self2, W_neigh2, b2):
    src = edge_index[0].reshape(NW, EPW)
    dst = edge_index[1].reshape(NW, EPW)
    pad = EPW_PAD - EPW
    src_p = jnp.pad(src, ((0, 0), (0, pad))).reshape(NC, NS, K, CH)
    dst_p = jnp.pad(dst, ((0, 0), (0, pad)), constant_values=N).reshape(NC, NS, K, CH)
    zf = jnp.zeros((NPAD, D), jnp.float32)
    ones = jnp.ones((CH, D), jnp.float32)
    b1r = b1.reshape(1, H)
    b2r = b2.reshape(1, H)

    cnt = _sc_cnt(dst_p, zf, ones)
    agg1 = _sc_agg(in_feat, src_p, dst_p, zf)
    c0 = cnt[0, :N, 0:1]
    c1 = cnt[1, :N, 0:1]
    h1 = _tc_layer(in_feat, agg1[0, :N], agg1[1, :N], c0, c1,
                   W_self1, W_neigh1, b1r, True)
    agg2 = _sc_agg(h1, src_p, dst_p, zf)
    out = _tc_layer(h1, agg2[0, :N], agg2[1, :N], c0, c1,
                    W_self2, W_neigh2, b2r, False)
    return out
